# baseline (speedup 1.0000x reference)
import jax
import jax.numpy as jnp
from jax import lax
from jax.experimental import pallas as pl
from jax.experimental.pallas import tpu as pltpu
from jax.experimental.pallas import tpu_sc as plsc

N_NODES = 10000
N_EDGES = 320000
HIDDEN = 128
NUM_LAYERS = 6
NUM_CLASSES = 40
N_GRAPHS = 64
INV_BN_STD = 1.0 / (1.0 + 1e-5) ** 0.5

NC, NS = 2, 16
NW = NC * NS
EPW = N_EDGES // NW
BLK = 125
NBLK = EPW // BLK
N_PAD = 10240
RPS = N_PAD // NS
CNT_W = 128
NSTG = NBLK // 4
NIT = NBLK // 8

SBLK = 80
SNBLK = 128
EPW_PAD = SBLK * SNBLK
NSTG2 = SNBLK // 4
NIT2 = SNBLK // 8

_sc_mesh = plsc.VectorSubcoreMesh(core_axis_name="c", subcore_axis_name="s",
                                  num_cores=NC, num_subcores=NS)


_DEG_WIN = 8


def _sc_deg_body(didx_hbm, ones_hbm, zeros_hbm, cnt_hbm, didx, ones_v, acc, sem):
    c = lax.axis_index("c")
    s = lax.axis_index("s")
    w = c * NS + s
    pltpu.sync_copy(didx_hbm.at[w], didx)
    pltpu.sync_copy(ones_hbm, ones_v)
    pltpu.sync_copy(zeros_hbm, acc.at[pl.ds(s * RPS, RPS)])
    plsc.subcore_barrier()

    def body(j, carry):
        pltpu.async_copy(ones_v, acc.at[didx.at[j]], sem, add=True)

        @pl.when(j >= _DEG_WIN)
        def _():
            pltpu.make_async_copy(ones_v, acc.at[didx.at[0]], sem).wait()

        return carry

    lax.fori_loop(0, NBLK, body, 0)

    def drain(j, carry):
        pltpu.make_async_copy(ones_v, acc.at[didx.at[0]], sem).wait()
        return carry

    lax.fori_loop(0, _DEG_WIN, drain, 0)
    plsc.subcore_barrier()
    pltpu.sync_copy(acc.at[pl.ds(s * RPS, RPS)], cnt_hbm.at[c, pl.ds(s * RPS, RPS)])


_deg_call = pl.kernel(
    _sc_deg_body,
    out_type=jax.ShapeDtypeStruct((NC, N_PAD, CNT_W), jnp.float32),
    mesh=_sc_mesh,
    scratch_types=[
        pltpu.VMEM((NBLK, BLK), jnp.int32),
        pltpu.VMEM((BLK, CNT_W), jnp.float32),
        pltpu.VMEM_SHARED((N_PAD, CNT_W), jnp.float32),
        pltpu.SemaphoreType.DMA,
    ],
)


def _sc_scatter_body(y_hbm, sidx_hbm, didx_hbm, zeros_hbm, p_hbm,
                     sring, dring, rows, acc,
                     g0, g1, g2, g3, s0, s1, s2, s3, r0, r1, d0, d1):
    c = lax.axis_index("c")
    s = lax.axis_index("s")
    w = c * NS + s
    gsem = (g0, g1, g2, g3)
    ssem = (s0, s1, s2, s3)
    rsem = (r0, r1)
    dsem = (d0, d1)
    pltpu.sync_copy(sidx_hbm.at[w, 0], sring.at[0])
    pltpu.sync_copy(didx_hbm.at[w, 0], dring.at[0])
    pltpu.async_copy(sidx_hbm.at[w, 1], sring.at[1], rsem[1])
    pltpu.async_copy(didx_hbm.at[w, 1], dring.at[1], dsem[1])
    pltpu.sync_copy(zeros_hbm, acc.at[pl.ds(s * RPS, RPS)])
    plsc.subcore_barrier()

    def fire_g(q, r, b):
        pltpu.async_copy(y_hbm.at[sring.at[q, r]], rows.at[b], gsem[b])

    def wait_g(b):
        pltpu.make_async_copy(y_hbm.at[sring.at[0, 0]], rows.at[b],
                              gsem[b]).wait()

    def fire_s(q, r, b):
        pltpu.async_copy(rows.at[b], acc.at[dring.at[q, r]], ssem[b], add=True)

    def wait_s(b):
        pltpu.make_async_copy(rows.at[b], acc.at[dring.at[0, 0]],
                              ssem[b]).wait()

    fire_g(0, 0, 0)
    fire_g(0, 1, 1)
    fire_g(0, 2, 2)

    def body(it, carry):
        for k in range(8):
            b = k % 4
            wait_g(b)
            if k == 0:
                @pl.when(it > 0)
                def _():
                    pltpu.make_async_copy(didx_hbm.at[w, 0], dring.at[0],
                                          dsem[0]).wait()
            if k == 4:
                pltpu.make_async_copy(didx_hbm.at[w, 0], dring.at[1],
                                      dsem[1]).wait()
            fire_s(k // 4, k % 4, b)
            if k == 0:
                @pl.when(it > 0)
                def _():
                    wait_s(3)
                    pltpu.async_copy(didx_hbm.at[w, 2 * it + 1], dring.at[1],
                                     dsem[1])
            else:
                wait_s((k - 1) % 4)
            if k == 1:
                pltpu.make_async_copy(sidx_hbm.at[w, 0], sring.at[1],
                                      rsem[1]).wait()
            if k == 3:
                @pl.when(it < NIT2 - 1)
                def _():
                    pltpu.async_copy(sidx_hbm.at[w, 2 * it + 2], sring.at[0],
                                     rsem[0])
            if k == 4:
                @pl.when(it < NIT2 - 1)
                def _():
                    pltpu.async_copy(didx_hbm.at[w, 2 * it + 2], dring.at[0],
                                     dsem[0])
            if k == 5:
                @pl.when(it < NIT2 - 1)
                def _():
                    pltpu.make_async_copy(sidx_hbm.at[w, 0], sring.at[0],
                                          rsem[0]).wait()
            if k < 5:
                fire_g(((k + 3) // 4) % 2, (k + 3) % 4, (k + 3) % 4)
            else:
                @pl.when(it < NIT2 - 1)
                def _():
                    fire_g(0, (k + 3) % 4, (k + 3) % 4)
            if k == 7:
                @pl.when(it < NIT2 - 1)
                def _():
                    pltpu.async_copy(sidx_hbm.at[w, 2 * it + 3], sring.at[1],
                                     rsem[1])
        return carry

    lax.fori_loop(0, NIT2, body, 0)
    wait_s(3)
    plsc.subcore_barrier()
    pltpu.sync_copy(acc.at[pl.ds(s * RPS, RPS)], p_hbm.at[c, pl.ds(s * RPS, RPS)])


_scatter_call = pl.kernel(
    _sc_scatter_body,
    out_type=jax.ShapeDtypeStruct((NC, N_PAD, HIDDEN), jnp.float32),
    mesh=_sc_mesh,
    scratch_types=[
        pltpu.VMEM((2, 4, SBLK), jnp.int32),
        pltpu.VMEM((2, 4, SBLK), jnp.int32),
        pltpu.VMEM((4, SBLK, HIDDEN), jnp.float32),
        pltpu.VMEM_SHARED((N_PAD, HIDDEN), jnp.float32),
    ] + [pltpu.SemaphoreType.DMA] * 12,
)


_ROWS_BLK = 2000


def _k0_body(x_ref, w_ref, cnt_ref, y_ref, dinv_ref):
    deg = cnt_ref[0, :, 0:1] + cnt_ref[1, :, 0:1] + 1.0
    dinv = lax.rsqrt(deg)
    z = jnp.dot(x_ref[...], w_ref[...], preferred_element_type=jnp.float32)
    y_ref[...] = z * dinv
    dinv_ref[...] = dinv


_k0_call = pl.pallas_call(
    _k0_body,
    grid=(N_NODES // _ROWS_BLK,),
    in_specs=[
        pl.BlockSpec((_ROWS_BLK, HIDDEN), lambda i: (i, 0)),
        pl.BlockSpec((HIDDEN, HIDDEN), lambda i: (0, 0)),
        pl.BlockSpec((NC, _ROWS_BLK, CNT_W), lambda i: (0, i, 0)),
    ],
    out_specs=[
        pl.BlockSpec((_ROWS_BLK, HIDDEN), lambda i: (i, 0)),
        pl.BlockSpec((_ROWS_BLK, 1), lambda i: (i, 0)),
    ],
    out_shape=[
        jax.ShapeDtypeStruct((N_NODES, HIDDEN), jnp.float32),
        jax.ShapeDtypeStruct((N_NODES, 1), jnp.float32),
    ],
)


def _kmid_body(p_ref, y_ref, jk_ref, dinv_ref, b_ref, g_ref, be_ref, w_ref,
               ynew_ref, jknew_ref):
    dinv = dinv_ref[...]
    agg = (p_ref[0] + p_ref[1] + y_ref[...]) * dinv
    scale = g_ref[...] * INV_BN_STD
    h = jnp.maximum((agg + b_ref[...]) * scale + be_ref[...], 0.0)
    jknew_ref[...] = jnp.maximum(jk_ref[...], h)
    ynew_ref[...] = jnp.dot(h, w_ref[...], preferred_element_type=jnp.float32) * dinv


_kmid_call = pl.pallas_call(
    _kmid_body,
    grid=(N_NODES // _ROWS_BLK,),
    in_specs=[
        pl.BlockSpec((NC, _ROWS_BLK, HIDDEN), lambda i: (0, i, 0)),
        pl.BlockSpec((_ROWS_BLK, HIDDEN), lambda i: (i, 0)),
        pl.BlockSpec((_ROWS_BLK, HIDDEN), lambda i: (i, 0)),
        pl.BlockSpec((_ROWS_BLK, 1), lambda i: (i, 0)),
        pl.BlockSpec((1, HIDDEN), lambda i: (0, 0)),
        pl.BlockSpec((1, HIDDEN), lambda i: (0, 0)),
        pl.BlockSpec((1, HIDDEN), lambda i: (0, 0)),
        pl.BlockSpec((HIDDEN, HIDDEN), lambda i: (0, 0)),
    ],
    out_specs=[
        pl.BlockSpec((_ROWS_BLK, HIDDEN), lambda i: (i, 0)),
        pl.BlockSpec((_ROWS_BLK, HIDDEN), lambda i: (i, 0)),
    ],
    out_shape=[
        jax.ShapeDtypeStruct((N_NODES, HIDDEN), jnp.float32),
        jax.ShapeDtypeStruct((N_NODES, HIDDEN), jnp.float32),
    ],
)


def _kfin_body(p_ref, y_ref, jk_ref, dinv_ref, b_ref, g_ref, be_ref,
               batch_ref, fcw_ref, fcb_ref, out_ref):
    dinv = dinv_ref[...]
    agg = (p_ref[0, :N_NODES] + p_ref[1, :N_NODES] + y_ref[...]) * dinv
    scale = g_ref[...] * INV_BN_STD
    h = jnp.maximum((agg + b_ref[...]) * scale + be_ref[...], 0.0)
    hjk = jnp.maximum(jk_ref[...], h)
    gid = lax.broadcasted_iota(jnp.int32, (N_NODES, N_GRAPHS), 1)
    onehot = (batch_ref[...] == gid).astype(jnp.float32)
    dn = (((0,), (0,)), ((), ()))
    sums = lax.dot_general(onehot, hjk, dn,
                           preferred_element_type=jnp.float32)
    counts = lax.dot_general(onehot, jnp.ones((N_NODES, 1), jnp.float32), dn,
                             preferred_element_type=jnp.float32)
    pooled = sums / jnp.maximum(counts, 1.0)
    out_ref[...] = jnp.dot(pooled, fcw_ref[...],
                           preferred_element_type=jnp.float32) + fcb_ref[...]


_kfin_call = pl.pallas_call(
    _kfin_body,
    out_shape=jax.ShapeDtypeStruct((N_GRAPHS, NUM_CLASSES), jnp.float32),
    compiler_params=pltpu.CompilerParams(vmem_limit_bytes=100 * 1024 * 1024),
)


def kernel(x, edge_index, batch, Ws, bs, gammas, betas, fcW, fcb):
    dst = edge_index[1].reshape(NW, NBLK, BLK)
    npad = EPW_PAD - EPW
    srcp = jnp.concatenate(
        [edge_index[0].reshape(NW, EPW),
         jnp.zeros((NW, npad), jnp.int32)], axis=1).reshape(NW, NSTG2, 4, SBLK)
    dstp = jnp.concatenate(
        [edge_index[1].reshape(NW, EPW),
         jnp.full((NW, npad), N_NODES, jnp.int32)], axis=1).reshape(NW, NSTG2, 4, SBLK)
    zeros_h = jnp.zeros((RPS, HIDDEN), jnp.float32)
    zeros_c = jnp.zeros((RPS, CNT_W), jnp.float32)
    ones_c = jnp.ones((BLK, CNT_W), jnp.float32)

    cnt = _deg_call(dst, ones_c, zeros_c)
    y, dinv = _k0_call(x, Ws[0], cnt)
    jk = jnp.zeros((N_NODES, HIDDEN), jnp.float32)
    for t in range(1, NUM_LAYERS):
        p = _scatter_call(y, srcp, dstp, zeros_h)
        y, jk = _kmid_call(p, y, jk, dinv, bs[t - 1:t], gammas[t - 1:t],
                           betas[t - 1:t], Ws[t])
    p = _scatter_call(y, srcp, dstp, zeros_h)
    out = _kfin_call(p, y, jk, dinv, bs[5:6], gammas[5:6], betas[5:6],
                     batch.reshape(N_NODES, 1), fcW,
                     fcb.reshape(1, NUM_CLASSES))
    return out


# speedup vs baseline: 2.7236x; 2.7236x over previous
import jax
import jax.numpy as jnp
from jax import lax
from jax.experimental import pallas as pl
from jax.experimental.pallas import tpu as pltpu
from jax.experimental.pallas import tpu_sc as plsc

N_NODES = 10000
N_EDGES = 320000
HIDDEN = 128
NUM_LAYERS = 6
NUM_CLASSES = 40
N_GRAPHS = 64
INV_BN_STD = 1.0 / (1.0 + 1e-5) ** 0.5

NC, NS = 2, 16
NW = NC * NS
EPW = N_EDGES // NW
BLK = 125
NBLK = EPW // BLK
N_PAD = 10240
RPS = N_PAD // NS
CNT_W = 128
NSTG = NBLK // 4
NIT = NBLK // 8

_sc_mesh = plsc.VectorSubcoreMesh(core_axis_name="c", subcore_axis_name="s",
                                  num_cores=NC, num_subcores=NS)


DEG_ROWS = EPW // 16
TBL_R = N_PAD // 128


def _sc_deg_body(didx_hbm, cnt_hbm, didx, tbl):
    c = lax.axis_index("c")
    s = lax.axis_index("s")
    w = c * NS + s
    pltpu.sync_copy(didx_hbm.at[w], didx)
    z = jnp.zeros((16,), jnp.float32)

    def zrow(r, carry):
        for l in range(8):
            tbl[r, pl.ds(l * 16, 16)] = z
        return carry

    lax.fori_loop(0, TBL_R, zrow, 0)
    ones = jnp.ones((16,), jnp.float32)
    m127 = jnp.full((16,), 127, jnp.int32)

    def scat(j, carry):
        v = didx[j]
        plsc.addupdate_scatter(tbl, [lax.shift_right_logical(v, 7),
                                     lax.bitwise_and(v, m127)], ones)
        return carry

    lax.fori_loop(0, DEG_ROWS, scat, 0)
    pltpu.sync_copy(tbl, cnt_hbm.at[w])


_deg_call = pl.kernel(
    _sc_deg_body,
    out_type=jax.ShapeDtypeStruct((NW, TBL_R, 128), jnp.float32),
    mesh=_sc_mesh,
    compiler_params=pltpu.CompilerParams(needs_layout_passes=False),
    scratch_types=[
        pltpu.VMEM((DEG_ROWS, 16), jnp.int32),
        pltpu.VMEM((TBL_R, 128), jnp.float32),
    ],
)


def _sc_scatter_body(y_hbm, sidx_hbm, didx_hbm, zeros_hbm, p_hbm,
                     didx, sring, rows, acc,
                     gsem0, gsem1, ssem0, ssem1, rsem0, rsem1):
    c = lax.axis_index("c")
    s = lax.axis_index("s")
    w = c * NS + s
    gsem = (gsem0, gsem1)
    ssem = (ssem0, ssem1)
    pltpu.sync_copy(didx_hbm.at[w], didx)
    pltpu.sync_copy(sidx_hbm.at[w, 0], sring.at[0])
    pltpu.async_copy(sidx_hbm.at[w, 1], sring.at[1], rsem1)
    pltpu.sync_copy(zeros_hbm, acc.at[pl.ds(s * RPS, RPS)])
    plsc.subcore_barrier()

    def wait_g(h):
        pltpu.make_async_copy(y_hbm.at[sring.at[0, 0]], rows.at[h],
                              gsem[h]).wait()

    def fire_s(j, h):
        pltpu.async_copy(rows.at[h], acc.at[didx.at[j]], ssem[h], add=True)

    def wait_s(h):
        pltpu.make_async_copy(rows.at[h], acc.at[didx.at[0]], ssem[h]).wait()

    pltpu.async_copy(y_hbm.at[sring.at[0, 0]], rows.at[0], gsem0)

    def body(it, carry):
        j0 = 8 * it
        for k in range(8):
            h = k % 2
            wait_g(h)
            fire_s(j0 + k, h)
            if k == 0:
                @pl.when(it > 0)
                def _():
                    wait_s(1)
            else:
                wait_s(1 - h)
            if k == 3:
                pltpu.make_async_copy(sidx_hbm.at[w, 0], sring.at[1],
                                      rsem1).wait()
            if k < 7:
                q, r = (k + 1) // 4, (k + 1) % 4
                pltpu.async_copy(y_hbm.at[sring.at[q, r]], rows.at[1 - h],
                                 gsem[1 - h])
            if k == 3:
                @pl.when(it < NIT - 1)
                def _():
                    pltpu.async_copy(sidx_hbm.at[w, 2 * it + 2], sring.at[0],
                                     rsem0)
            if k == 7:
                @pl.when(it < NIT - 1)
                def _():
                    pltpu.make_async_copy(sidx_hbm.at[w, 0], sring.at[0],
                                          rsem0).wait()
                    pltpu.async_copy(y_hbm.at[sring.at[0, 0]], rows.at[0],
                                     gsem0)
                    pltpu.async_copy(sidx_hbm.at[w, 2 * it + 3], sring.at[1],
                                     rsem1)
        return carry

    lax.fori_loop(0, NIT, body, 0)
    wait_s(1)
    plsc.subcore_barrier()
    pltpu.sync_copy(acc.at[pl.ds(s * RPS, RPS)], p_hbm.at[c, pl.ds(s * RPS, RPS)])


_scatter_call = pl.kernel(
    _sc_scatter_body,
    out_type=jax.ShapeDtypeStruct((NC, N_PAD, HIDDEN), jnp.float32),
    mesh=_sc_mesh,
    scratch_types=[
        pltpu.VMEM((NBLK, BLK), jnp.int32),
        pltpu.VMEM((2, 4, BLK), jnp.int32),
        pltpu.VMEM((2, BLK, HIDDEN), jnp.float32),
        pltpu.VMEM_SHARED((N_PAD, HIDDEN), jnp.float32),
        pltpu.SemaphoreType.DMA,
        pltpu.SemaphoreType.DMA,
        pltpu.SemaphoreType.DMA,
        pltpu.SemaphoreType.DMA,
        pltpu.SemaphoreType.DMA,
        pltpu.SemaphoreType.DMA,
    ],
)


_ROWS_BLK = 2000


_K0_BLK = 2048


def _k0_body(x_ref, w_ref, cnt_ref, y_ref, dinv_ref):
    deg = jnp.sum(cnt_ref[...], axis=0) + 1.0
    dinv16 = lax.rsqrt(deg)
    bsel = (lax.broadcasted_iota(jnp.int32, (_K0_BLK, 16), 0) // 128 ==
            lax.broadcasted_iota(jnp.int32, (_K0_BLK, 16), 1))
    csel = (lax.broadcasted_iota(jnp.int32, (_K0_BLK, 128), 0) % 128 ==
            lax.broadcasted_iota(jnp.int32, (_K0_BLK, 128), 1))
    spread = jnp.dot(bsel.astype(jnp.float32), dinv16,
                     preferred_element_type=jnp.float32)
    dinv = jnp.sum(jnp.where(csel, spread, 0.0), axis=1, keepdims=True)
    z = jnp.dot(x_ref[...], w_ref[...], preferred_element_type=jnp.float32)
    y_ref[...] = z * dinv
    dinv_ref[...] = dinv


_k0_call = pl.pallas_call(
    _k0_body,
    grid=(pl.cdiv(N_NODES, _K0_BLK),),
    in_specs=[
        pl.BlockSpec((_K0_BLK, HIDDEN), lambda i: (i, 0)),
        pl.BlockSpec((HIDDEN, HIDDEN), lambda i: (0, 0)),
        pl.BlockSpec((NW, _K0_BLK // 128, 128), lambda i: (0, i, 0)),
    ],
    out_specs=[
        pl.BlockSpec((_K0_BLK, HIDDEN), lambda i: (i, 0)),
        pl.BlockSpec((_K0_BLK, 1), lambda i: (i, 0)),
    ],
    out_shape=[
        jax.ShapeDtypeStruct((N_NODES, HIDDEN), jnp.float32),
        jax.ShapeDtypeStruct((N_NODES, 1), jnp.float32),
    ],
)


def _kmid_body(p_ref, y_ref, jk_ref, dinv_ref, b_ref, g_ref, be_ref, w_ref,
               ynew_ref, jknew_ref):
    dinv = dinv_ref[...]
    agg = (p_ref[0] + p_ref[1] + y_ref[...]) * dinv
    scale = g_ref[...] * INV_BN_STD
    h = jnp.maximum((agg + b_ref[...]) * scale + be_ref[...], 0.0)
    jknew_ref[...] = jnp.maximum(jk_ref[...], h)
    ynew_ref[...] = jnp.dot(h, w_ref[...], preferred_element_type=jnp.float32) * dinv


_kmid_call = pl.pallas_call(
    _kmid_body,
    grid=(N_NODES // _ROWS_BLK,),
    in_specs=[
        pl.BlockSpec((NC, _ROWS_BLK, HIDDEN), lambda i: (0, i, 0)),
        pl.BlockSpec((_ROWS_BLK, HIDDEN), lambda i: (i, 0)),
        pl.BlockSpec((_ROWS_BLK, HIDDEN), lambda i: (i, 0)),
        pl.BlockSpec((_ROWS_BLK, 1), lambda i: (i, 0)),
        pl.BlockSpec((1, HIDDEN), lambda i: (0, 0)),
        pl.BlockSpec((1, HIDDEN), lambda i: (0, 0)),
        pl.BlockSpec((1, HIDDEN), lambda i: (0, 0)),
        pl.BlockSpec((HIDDEN, HIDDEN), lambda i: (0, 0)),
    ],
    out_specs=[
        pl.BlockSpec((_ROWS_BLK, HIDDEN), lambda i: (i, 0)),
        pl.BlockSpec((_ROWS_BLK, HIDDEN), lambda i: (i, 0)),
    ],
    out_shape=[
        jax.ShapeDtypeStruct((N_NODES, HIDDEN), jnp.float32),
        jax.ShapeDtypeStruct((N_NODES, HIDDEN), jnp.float32),
    ],
)


def _kfin_body(p_ref, y_ref, jk_ref, dinv_ref, b_ref, g_ref, be_ref,
               batch_ref, fcw_ref, fcb_ref, out_ref):
    dinv = dinv_ref[...]
    agg = (p_ref[0, :N_NODES] + p_ref[1, :N_NODES] + y_ref[...]) * dinv
    scale = g_ref[...] * INV_BN_STD
    h = jnp.maximum((agg + b_ref[...]) * scale + be_ref[...], 0.0)
    hjk = jnp.maximum(jk_ref[...], h)
    gid = lax.broadcasted_iota(jnp.int32, (N_NODES, N_GRAPHS), 1)
    onehot = (batch_ref[...] == gid).astype(jnp.float32)
    dn = (((0,), (0,)), ((), ()))
    sums = lax.dot_general(onehot, hjk, dn,
                           preferred_element_type=jnp.float32)
    counts = lax.dot_general(onehot, jnp.ones((N_NODES, 1), jnp.float32), dn,
                             preferred_element_type=jnp.float32)
    pooled = sums / jnp.maximum(counts, 1.0)
    out_ref[...] = jnp.dot(pooled, fcw_ref[...],
                           preferred_element_type=jnp.float32) + fcb_ref[...]


_kfin_call = pl.pallas_call(
    _kfin_body,
    out_shape=jax.ShapeDtypeStruct((N_GRAPHS, NUM_CLASSES), jnp.float32),
    compiler_params=pltpu.CompilerParams(vmem_limit_bytes=100 * 1024 * 1024),
)


def kernel(x, edge_index, batch, Ws, bs, gammas, betas, fcW, fcb):
    src = edge_index[0].reshape(NW, NSTG, 4, BLK)
    dst = edge_index[1].reshape(NW, NBLK, BLK)
    dst16 = edge_index[1].reshape(NW, DEG_ROWS, 16)
    zeros_h = jnp.zeros((RPS, HIDDEN), jnp.float32)

    cnt = _deg_call(dst16)
    y, dinv = _k0_call(x, Ws[0], cnt)
    jk = jnp.zeros((N_NODES, HIDDEN), jnp.float32)
    for t in range(1, NUM_LAYERS):
        p = _scatter_call(y, src, dst, zeros_h)
        y, jk = _kmid_call(p, y, jk, dinv, bs[t - 1:t], gammas[t - 1:t],
                           betas[t - 1:t], Ws[t])
    p = _scatter_call(y, src, dst, zeros_h)
    out = _kfin_call(p, y, jk, dinv, bs[5:6], gammas[5:6], betas[5:6],
                     batch.reshape(N_NODES, 1), fcW,
                     fcb.reshape(1, NUM_CLASSES))
    return out
